# baseline (speedup 1.0000x reference)
import functools

import jax
import jax.numpy as jnp
from jax import lax
from jax.experimental import pallas as pl
from jax.experimental.pallas import tpu as pltpu
from jax.experimental.pallas import tpu_sc as plsc

_NEMB = 4
_NFEATS = 15
_NHIDDEN = 128
_NEWDIM = 3 * _NEMB + _NEMB + _NEMB + _NFEATS
_NTAB = 5
_EPS = 1e-5

_B = 16384
_NW = 32
_CHUNK = _B // _NW
_NCODES = 5 ** 5
_TABW = 32


def _sc_gather_body(ints_hbm, tab_hbm, out_hbm, raw_v, code_v, rows_v, sem):
    wid = lax.axis_index("s") * 2 + lax.axis_index("c")
    base = wid * _CHUNK
    pltpu.sync_copy(ints_hbm.at[:, pl.ds(base, _CHUNK)], raw_v)

    def grp(g, carry):
        sl = pl.ds(g * 16, 16)
        code = (raw_v[0, sl] + 5 * raw_v[1, sl] + 25 * raw_v[2, sl]
                + 125 * raw_v[3, sl] + 625 * raw_v[4, sl])
        code_v[g // 8, pl.ds((g % 8) * 16, 16)] = code
        return carry

    lax.fori_loop(0, _CHUNK // 16, grp, 0)
    for j in range(_CHUNK // 128):
        pltpu.async_copy(tab_hbm.at[code_v.at[j]],
                         rows_v.at[pl.ds(j * 128, 128)], sem).wait()
    pltpu.sync_copy(rows_v, out_hbm.at[pl.ds(base, _CHUNK)])


def _sc_gather(ints_flat, tcomb):
    gather = pl.kernel(
        _sc_gather_body,
        out_type=jax.ShapeDtypeStruct((_B, _TABW), jnp.float32),
        mesh=plsc.VectorSubcoreMesh(core_axis_name="c",
                                    subcore_axis_name="s"),
        scratch_types=[
            pltpu.VMEM((5, _CHUNK), jnp.int32),
            pltpu.VMEM((_CHUNK // 128, 128), jnp.int32),
            pltpu.VMEM((_CHUNK, _TABW), jnp.float32),
            pltpu.SemaphoreType.DMA,
        ],
        compiler_params=pltpu.CompilerParams(use_tc_tiling_on_sc=False),
    )
    return gather(ints_flat, tcomb)


def _board_kernel(embT_ref, featsT_ref, waug_ref, out_ref):
    R = out_ref.shape[0]
    embT = embT_ref[: 4 * _NTAB, :]
    featsT = featsT_ref[...]
    combT = jnp.concatenate([embT, featsT], axis=0)

    mu = jnp.mean(combT, axis=0, keepdims=True)
    xm = combT - mu
    var = jnp.mean(xm * xm, axis=0, keepdims=True)
    rs = lax.rsqrt(var + _EPS)
    norm2 = jnp.concatenate([xm * rs, jnp.ones((1, R), jnp.float32)],
                            axis=0)

    y = lax.dot_general(norm2, waug_ref[...],
                        dimension_numbers=(((0,), (0,)), ((), ())),
                        preferred_element_type=jnp.float32)
    out_ref[...] = jnp.maximum(y, 0.0)


@functools.partial(jax.jit, static_argnames=("block_r",))
def _run(boardInts, boardFeats, twEmb, trEmb, weatherEmb, terrainEmb,
         ln_g, ln_b, W, b, block_r=8192):
    B = boardInts.shape[0]
    featsT = boardFeats.T

    i = jnp.arange(_NCODES)
    v0, v1 = i % 5, (i // 5) % 5
    v2, v3, v4 = (i // 25) % 5, (i // 125) % 5, (i // 625) % 5
    tcomb = jnp.concatenate(
        [twEmb[v0], twEmb[v1], trEmb[v2], weatherEmb[v3], terrainEmb[v4],
         jnp.zeros((_NCODES, _TABW - 4 * _NTAB), jnp.float32)], axis=1)

    emb32 = _sc_gather(boardInts.T, tcomb)
    embT = emb32.T

    waug = jnp.concatenate(
        [ln_g[:, None] * W, (ln_b @ W + b)[None, :]], axis=0)

    grid = (B // block_r,)
    full = lambda shape: pl.BlockSpec(shape, lambda i: (0,) * len(shape))
    return pl.pallas_call(
        _board_kernel,
        grid=grid,
        in_specs=[
            pl.BlockSpec((_TABW, block_r), lambda i: (0, i)),
            pl.BlockSpec((_NFEATS, block_r), lambda i: (0, i)),
            full((_NEWDIM + 1, _NHIDDEN)),
        ],
        out_specs=pl.BlockSpec((block_r, _NHIDDEN), lambda i: (i, 0)),
        out_shape=jax.ShapeDtypeStruct((B, _NHIDDEN), jnp.float32),
    )(embT, featsT, waug)


def kernel(boardInts, boardFeats, twEmb, trEmb, weatherEmb, terrainEmb,
           ln_g, ln_b, W, b):
    return _run(boardInts, boardFeats, twEmb, trEmb, weatherEmb, terrainEmb,
                ln_g, ln_b, W, b)


# speedup vs baseline: 6.4734x; 6.4734x over previous
import functools

import jax
import jax.numpy as jnp
from jax import lax
from jax.experimental import pallas as pl

_NEMB = 4
_NFEATS = 15
_NHIDDEN = 128
_NEWDIM = 3 * _NEMB + _NEMB + _NEMB + _NFEATS
_NTAB = 5
_EPS = 1e-5


def _board_kernel(intsT_ref, featsT_ref, gmap_ref, waug_ref, out_ref):
    R = out_ref.shape[0]
    intsT = intsT_ref[...]
    featsT = featsT_ref[...]

    rep = jnp.concatenate([intsT] * _NTAB, axis=0)
    val = lax.broadcasted_iota(jnp.int32, (5 * _NTAB, 1), 0) // _NTAB
    ohT = (rep == val).astype(jnp.float32)

    embT = jnp.dot(gmap_ref[...], ohT,
                   preferred_element_type=jnp.float32)
    combT = jnp.concatenate([embT, featsT], axis=0)

    mu = jnp.mean(combT, axis=0, keepdims=True)
    xm = combT - mu
    var = jnp.mean(xm * xm, axis=0, keepdims=True)
    rs = lax.rsqrt(var + _EPS)
    norm2 = jnp.concatenate([xm * rs, jnp.ones((1, R), jnp.float32)],
                            axis=0)

    y = lax.dot_general(norm2, waug_ref[...],
                        dimension_numbers=(((0,), (0,)), ((), ())),
                        preferred_element_type=jnp.float32)
    out_ref[...] = jnp.maximum(y, 0.0)


@functools.partial(jax.jit, static_argnames=("block_r",))
def _run(boardInts, boardFeats, twEmb, trEmb, weatherEmb, terrainEmb,
         ln_g, ln_b, W, b, block_r=8192):
    B = boardInts.shape[0]
    intsT = boardInts.T
    featsT = boardFeats.T

    tables = jnp.stack([twEmb, twEmb, trEmb, weatherEmb, terrainEmb])
    t_ckv = jnp.transpose(tables, (0, 2, 1))
    gmap = (t_ckv[:, :, :, None] * jnp.eye(_NTAB, dtype=jnp.float32)[:, None, None, :]
            ).reshape(4 * _NTAB, 5 * _NTAB)

    waug = jnp.concatenate(
        [ln_g[:, None] * W, (ln_b @ W + b)[None, :]], axis=0)

    grid = (B // block_r,)
    full = lambda shape: pl.BlockSpec(shape, lambda i: (0,) * len(shape))
    return pl.pallas_call(
        _board_kernel,
        grid=grid,
        in_specs=[
            pl.BlockSpec((5, block_r), lambda i: (0, i)),
            pl.BlockSpec((_NFEATS, block_r), lambda i: (0, i)),
            full((4 * _NTAB, 5 * _NTAB)),
            full((_NEWDIM + 1, _NHIDDEN)),
        ],
        out_specs=pl.BlockSpec((block_r, _NHIDDEN), lambda i: (i, 0)),
        out_shape=jax.ShapeDtypeStruct((B, _NHIDDEN), jnp.float32),
    )(intsT, featsT, gmap, waug)


def kernel(boardInts, boardFeats, twEmb, trEmb, weatherEmb, terrainEmb,
           ln_g, ln_b, W, b):
    return _run(boardInts, boardFeats, twEmb, trEmb, weatherEmb, terrainEmb,
                ln_g, ln_b, W, b)


# speedup vs baseline: 6.8098x; 1.0520x over previous
import functools

import jax
import jax.numpy as jnp
from jax import lax
from jax.experimental import pallas as pl

_NEMB = 4
_NFEATS = 15
_NHIDDEN = 128
_NEWDIM = 3 * _NEMB + _NEMB + _NEMB + _NFEATS
_NTAB = 5
_EPS = 1e-5


def _board_kernel(intsT_ref, featsT_ref, gmap_ref, waug_ref, out_ref):
    R = out_ref.shape[0]
    intsT = intsT_ref[...]
    featsT = featsT_ref[...]

    rep = jnp.concatenate([intsT] * _NTAB, axis=0)
    val = lax.broadcasted_iota(jnp.int32, (5 * _NTAB, 1), 0) // _NTAB
    ohT = (rep == val).astype(jnp.float32)

    embT = jnp.dot(gmap_ref[...], ohT,
                   preferred_element_type=jnp.float32)
    combT = jnp.concatenate([embT, featsT], axis=0)

    mu = jnp.mean(combT, axis=0, keepdims=True)
    xm = combT - mu
    var = jnp.mean(xm * xm, axis=0, keepdims=True)
    rs = lax.rsqrt(var + _EPS)
    norm2 = jnp.concatenate([xm * rs, jnp.ones((1, R), jnp.float32)],
                            axis=0)

    y = lax.dot_general(norm2.astype(jnp.bfloat16),
                        waug_ref[...].astype(jnp.bfloat16),
                        dimension_numbers=(((0,), (0,)), ((), ())),
                        preferred_element_type=jnp.float32)
    out_ref[...] = jnp.maximum(y, 0.0)


@functools.partial(jax.jit, static_argnames=("block_r",))
def _run(boardInts, boardFeats, twEmb, trEmb, weatherEmb, terrainEmb,
         ln_g, ln_b, W, b, block_r=8192):
    B = boardInts.shape[0]
    intsT = boardInts.T
    featsT = boardFeats.T

    tables = jnp.stack([twEmb, twEmb, trEmb, weatherEmb, terrainEmb])
    t_ckv = jnp.transpose(tables, (0, 2, 1))
    gmap = (t_ckv[:, :, :, None] * jnp.eye(_NTAB, dtype=jnp.float32)[:, None, None, :]
            ).reshape(4 * _NTAB, 5 * _NTAB)

    waug = jnp.concatenate(
        [ln_g[:, None] * W, (ln_b @ W + b)[None, :]], axis=0)

    grid = (B // block_r,)
    full = lambda shape: pl.BlockSpec(shape, lambda i: (0,) * len(shape))
    return pl.pallas_call(
        _board_kernel,
        grid=grid,
        in_specs=[
            pl.BlockSpec((5, block_r), lambda i: (0, i)),
            pl.BlockSpec((_NFEATS, block_r), lambda i: (0, i)),
            full((4 * _NTAB, 5 * _NTAB)),
            full((_NEWDIM + 1, _NHIDDEN)),
        ],
        out_specs=pl.BlockSpec((block_r, _NHIDDEN), lambda i: (i, 0)),
        out_shape=jax.ShapeDtypeStruct((B, _NHIDDEN), jnp.float32),
    )(intsT, featsT, gmap, waug)


def kernel(boardInts, boardFeats, twEmb, trEmb, weatherEmb, terrainEmb,
           ln_g, ln_b, W, b):
    return _run(boardInts, boardFeats, twEmb, trEmb, weatherEmb, terrainEmb,
                ln_g, ln_b, W, b)


# speedup vs baseline: 7.7019x; 1.1310x over previous
import functools

import jax
import jax.numpy as jnp
from jax import lax
from jax.experimental import pallas as pl
from jax.experimental.pallas import tpu as pltpu

_NEMB = 4
_NFEATS = 15
_NHIDDEN = 128
_NEWDIM = 3 * _NEMB + _NEMB + _NEMB + _NFEATS
_NTAB = 5
_EPS = 1e-5


def _board_kernel(intsT_ref, featsT_ref, gmap_ref, waug_ref, out_ref):
    R = out_ref.shape[0]
    intsT = intsT_ref[...]
    featsT = featsT_ref[...]

    rep = jnp.concatenate([intsT] * _NTAB, axis=0)
    val = lax.broadcasted_iota(jnp.int32, (5 * _NTAB, 1), 0) // _NTAB
    ohT = (rep == val).astype(jnp.float32)

    embT = jnp.dot(gmap_ref[...], ohT,
                   preferred_element_type=jnp.float32)
    combT = jnp.concatenate([embT, featsT], axis=0)

    mu = jnp.mean(combT, axis=0, keepdims=True)
    xm = combT - mu
    var = jnp.mean(xm * xm, axis=0, keepdims=True)
    rs = lax.rsqrt(var + _EPS)
    norm2 = jnp.concatenate([xm * rs, jnp.ones((1, R), jnp.float32)],
                            axis=0)

    y = lax.dot_general(norm2.astype(jnp.bfloat16),
                        waug_ref[...].astype(jnp.bfloat16),
                        dimension_numbers=(((0,), (0,)), ((), ())),
                        preferred_element_type=jnp.float32)
    out_ref[...] = jnp.maximum(y, 0.0)


@functools.partial(jax.jit, static_argnames=("block_r",))
def _run(boardInts, boardFeats, twEmb, trEmb, weatherEmb, terrainEmb,
         ln_g, ln_b, W, b, block_r=8192):
    B = boardInts.shape[0]
    intsT = boardInts.T
    featsT = boardFeats.T

    tables = jnp.stack([twEmb, twEmb, trEmb, weatherEmb, terrainEmb])
    t_ckv = jnp.transpose(tables, (0, 2, 1))
    gmap = (t_ckv[:, :, :, None] * jnp.eye(_NTAB, dtype=jnp.float32)[:, None, None, :]
            ).reshape(4 * _NTAB, 5 * _NTAB)

    waug = jnp.concatenate(
        [ln_g[:, None] * W, (ln_b @ W + b)[None, :]], axis=0)

    grid = (B // block_r,)
    full = lambda shape: pl.BlockSpec(shape, lambda i: (0,) * len(shape))
    return pl.pallas_call(
        _board_kernel,
        grid=grid,
        in_specs=[
            pl.BlockSpec((5, block_r), lambda i: (0, i)),
            pl.BlockSpec((_NFEATS, block_r), lambda i: (0, i)),
            full((4 * _NTAB, 5 * _NTAB)),
            full((_NEWDIM + 1, _NHIDDEN)),
        ],
        out_specs=pl.BlockSpec((block_r, _NHIDDEN), lambda i: (i, 0)),
        out_shape=jax.ShapeDtypeStruct((B, _NHIDDEN), jnp.float32),
        compiler_params=pltpu.CompilerParams(
            allow_input_fusion=[True, True, False, False]),
    )(intsT, featsT, gmap, waug)


def kernel(boardInts, boardFeats, twEmb, trEmb, weatherEmb, terrainEmb,
           ln_g, ln_b, W, b):
    return _run(boardInts, boardFeats, twEmb, trEmb, weatherEmb, terrainEmb,
                ln_g, ln_b, W, b)
